# baseline (speedup 1.0000x reference)
import functools

import jax
import jax.numpy as jnp
from jax.experimental import pallas as pl
from jax.experimental.pallas import tpu as pltpu

DIM = 128
C = 32
CP1 = C + 1

_INTERPRET = False


def _branch_body(o_ref, adj_ref, a_ref, w_ref, v_ref, gate_ref, out_ref,
                 vw_s, h_s, *, bb):
    o = o_ref[0]
    w = w_ref[0]
    adj = adj_ref[0]
    o_w = jnp.dot(o, w, preferred_element_type=jnp.float32)
    adj_w = jnp.dot(adj.reshape(bb * C, DIM), w,
                    preferred_element_type=jnp.float32)
    vw_s[:, 0:1, :] = o_w[:, None, :]
    vw_s[:, 1:, :] = adj_w.reshape(bb, C, DIM)

    def bmm_step(b, carry):
        a_b = a_ref[0, b]
        s = jax.lax.dot_general(a_b, vw_s[b], (((1,), (0,)), ((), ())),
                                preferred_element_type=jnp.float32)
        h_s[b] = jnp.maximum(s, 0.0)
        return carry

    jax.lax.fori_loop(0, bb, bmm_step, 0)

    h = h_s[...]
    tmp = jnp.maximum(h * o[:, None, :], 0.0)
    v = v_ref[0, 0]
    score = jnp.sum(tmp * v[None, None, :], axis=2)
    m = jnp.max(score, axis=1, keepdims=True)
    e = jnp.exp(score - m)
    alpha = e / jnp.sum(e, axis=1, keepdims=True)
    sg = jnp.sum(alpha[:, :, None] * h, axis=1)
    g = jax.nn.sigmoid(gate_ref[0, 0])
    out_ref[0] = g[None, :] * o + (1.0 - g[None, :]) * sg


def _branches(o_stack, adj_stack, a_stack, w_pair, v_pair, gate_pair, *, bb):
    nbr, batch = o_stack.shape[0], o_stack.shape[1]
    grid = (nbr, batch // bb)
    return pl.pallas_call(
        functools.partial(_branch_body, bb=bb),
        grid=grid,
        in_specs=[
            pl.BlockSpec((1, bb, DIM), lambda b, g: (b, g, 0)),
            pl.BlockSpec((1, bb, C, DIM), lambda b, g: (b, g, 0, 0)),
            pl.BlockSpec((1, bb, CP1, CP1), lambda b, g: (b, g, 0, 0)),
            pl.BlockSpec((1, DIM, DIM), lambda b, g: (b // 4, 0, 0)),
            pl.BlockSpec((1, 1, DIM), lambda b, g: (b // 4, 0, 0)),
            pl.BlockSpec((1, 1, DIM), lambda b, g: (b // 4, 0, 0)),
        ],
        out_specs=pl.BlockSpec((1, bb, DIM), lambda b, g: (b, g, 0)),
        out_shape=jax.ShapeDtypeStruct((nbr, batch, DIM), jnp.float32),
        scratch_shapes=[
            pltpu.VMEM((bb, CP1, DIM), jnp.float32),
            pltpu.VMEM((bb, CP1, DIM), jnp.float32),
        ],
        interpret=_INTERPRET,
    )(o_stack, adj_stack, a_stack, w_pair, v_pair, gate_pair)


def _score_body(o_ref, p_ref, n_ref):
    x = o_ref[...]
    p = x[0] + x[4] - x[1]
    n = x[2] + x[5] - x[3]
    p_ref[...] = jnp.sum(jnp.abs(p), axis=1)
    n_ref[...] = jnp.sum(jnp.abs(n), axis=1)


def _scores(o_out, *, bb):
    batch = o_out.shape[1]
    grid = (batch // bb,)
    return pl.pallas_call(
        _score_body,
        grid=grid,
        in_specs=[pl.BlockSpec((6, bb, DIM), lambda g: (0, g, 0))],
        out_specs=[
            pl.BlockSpec((bb,), lambda g: (g,)),
            pl.BlockSpec((bb,), lambda g: (g,)),
        ],
        out_shape=[
            jax.ShapeDtypeStruct((batch,), jnp.float32),
            jax.ShapeDtypeStruct((batch,), jnp.float32),
        ],
        interpret=_INTERPRET,
    )(o_out)


def kernel(epoch, pos_h, pos_r, pos_t, neg_h, neg_r, neg_t,
           ph_A, pr_A, pt_A, nh_A, nr_A, nt_A,
           ph_ctx, pt_ctx, nh_ctx, nt_ctx, pr_ctx, nr_ctx,
           entity_emb, relation_emb, entity_context, relation_context,
           entity_gcn_weight, relation_gcn_weight,
           gate_entity, gate_relation, v_ent, v_rel):
    batch = pos_h.shape[0]

    def tk(tab, idx):
        return jnp.take(tab, idx.astype(jnp.int32), axis=0)

    o_stack = jnp.stack([
        tk(entity_emb, pos_h), tk(entity_emb, pos_t),
        tk(entity_emb, neg_h), tk(entity_emb, neg_t),
        tk(relation_emb, pos_r), tk(relation_emb, neg_r)])

    def rel_adj(ctx):
        a = tk(relation_context, ctx)
        return a.reshape(batch, C, 2, DIM).sum(axis=2)

    adj_stack = jnp.stack([
        tk(entity_context, ph_ctx), tk(entity_context, pt_ctx),
        tk(entity_context, nh_ctx), tk(entity_context, nt_ctx),
        rel_adj(pr_ctx), rel_adj(nr_ctx)])

    a_stack = jnp.stack([ph_A, pt_A, nh_A, nt_A, pr_A, nr_A])
    w_pair = jnp.stack([entity_gcn_weight, relation_gcn_weight])
    v_pair = jnp.stack([v_ent, v_rel]).reshape(2, 1, DIM)
    gate_pair = jnp.stack([gate_entity, gate_relation]).reshape(2, 1, DIM)

    bb = min(128, batch)
    o_out = _branches(o_stack, adj_stack, a_stack, w_pair, v_pair, gate_pair,
                      bb=bb)
    p_score, n_score = _scores(o_out, bb=min(1024, batch))
    return p_score, n_score


# speedup vs baseline: 1.7990x; 1.7990x over previous
import functools

import jax
import jax.numpy as jnp
from jax.experimental import pallas as pl
from jax.experimental.pallas import tpu as pltpu

DIM = 128
C = 32
CP1 = C + 1

_INTERPRET = False


def _branch_body(o_ref, adj_ref, a_ref, w_ref, v_ref, gate_ref, out_ref,
                 *, bb):
    o = o_ref[0]
    w = w_ref[0]
    adj = adj_ref[0]
    o_w = jnp.dot(o, w, preferred_element_type=jnp.float32)
    adj_w = jnp.dot(adj.reshape(bb * C, DIM), w,
                    preferred_element_type=jnp.float32)
    vw3 = jnp.concatenate([o_w[:, None, :], adj_w.reshape(bb, C, DIM)],
                          axis=1)
    a3 = a_ref[0]
    s3 = jax.lax.dot_general(a3, vw3, (((2,), (1,)), ((0,), (0,))),
                             preferred_element_type=jnp.float32)
    h = jnp.maximum(s3, 0.0)
    tmp = jnp.maximum(h * o[:, None, :], 0.0)
    v = v_ref[0, 0]
    score = jnp.sum(tmp * v[None, None, :], axis=2)
    m = jnp.max(score, axis=1, keepdims=True)
    e = jnp.exp(score - m)
    alpha = e / jnp.sum(e, axis=1, keepdims=True)
    sg = jnp.sum(alpha[:, :, None] * h, axis=1)
    g = jax.nn.sigmoid(gate_ref[0, 0])
    out_ref[0] = g[None, :] * o + (1.0 - g[None, :]) * sg


def _branches(o_stack, adj_stack, a_stack, w_pair, v_pair, gate_pair, *, bb):
    nbr, batch = o_stack.shape[0], o_stack.shape[1]
    grid = (nbr, batch // bb)
    return pl.pallas_call(
        functools.partial(_branch_body, bb=bb),
        grid=grid,
        in_specs=[
            pl.BlockSpec((1, bb, DIM), lambda b, g: (b, g, 0)),
            pl.BlockSpec((1, bb, C, DIM), lambda b, g: (b, g, 0, 0)),
            pl.BlockSpec((1, bb, CP1, CP1), lambda b, g: (b, g, 0, 0)),
            pl.BlockSpec((1, DIM, DIM), lambda b, g: (b // 4, 0, 0)),
            pl.BlockSpec((1, 1, DIM), lambda b, g: (b // 4, 0, 0)),
            pl.BlockSpec((1, 1, DIM), lambda b, g: (b // 4, 0, 0)),
        ],
        out_specs=pl.BlockSpec((1, bb, DIM), lambda b, g: (b, g, 0)),
        out_shape=jax.ShapeDtypeStruct((nbr, batch, DIM), jnp.float32),
        interpret=_INTERPRET,
    )(o_stack, adj_stack, a_stack, w_pair, v_pair, gate_pair)


def _score_body(o_ref, p_ref, n_ref):
    x = o_ref[...]
    p = x[0] + x[4] - x[1]
    n = x[2] + x[5] - x[3]
    p_ref[...] = jnp.sum(jnp.abs(p), axis=1)
    n_ref[...] = jnp.sum(jnp.abs(n), axis=1)


def _scores(o_out, *, bb):
    batch = o_out.shape[1]
    grid = (batch // bb,)
    return pl.pallas_call(
        _score_body,
        grid=grid,
        in_specs=[pl.BlockSpec((6, bb, DIM), lambda g: (0, g, 0))],
        out_specs=[
            pl.BlockSpec((bb,), lambda g: (g,)),
            pl.BlockSpec((bb,), lambda g: (g,)),
        ],
        out_shape=[
            jax.ShapeDtypeStruct((batch,), jnp.float32),
            jax.ShapeDtypeStruct((batch,), jnp.float32),
        ],
        interpret=_INTERPRET,
    )(o_out)


def kernel(epoch, pos_h, pos_r, pos_t, neg_h, neg_r, neg_t,
           ph_A, pr_A, pt_A, nh_A, nr_A, nt_A,
           ph_ctx, pt_ctx, nh_ctx, nt_ctx, pr_ctx, nr_ctx,
           entity_emb, relation_emb, entity_context, relation_context,
           entity_gcn_weight, relation_gcn_weight,
           gate_entity, gate_relation, v_ent, v_rel):
    batch = pos_h.shape[0]

    def tk(tab, idx):
        return jnp.take(tab, idx.astype(jnp.int32), axis=0)

    o_stack = jnp.stack([
        tk(entity_emb, pos_h), tk(entity_emb, pos_t),
        tk(entity_emb, neg_h), tk(entity_emb, neg_t),
        tk(relation_emb, pos_r), tk(relation_emb, neg_r)])

    def rel_adj(ctx):
        a = tk(relation_context, ctx)
        return a.reshape(batch, C, 2, DIM).sum(axis=2)

    adj_stack = jnp.stack([
        tk(entity_context, ph_ctx), tk(entity_context, pt_ctx),
        tk(entity_context, nh_ctx), tk(entity_context, nt_ctx),
        rel_adj(pr_ctx), rel_adj(nr_ctx)])

    a_stack = jnp.stack([ph_A, pt_A, nh_A, nt_A, pr_A, nr_A])
    w_pair = jnp.stack([entity_gcn_weight, relation_gcn_weight])
    v_pair = jnp.stack([v_ent, v_rel]).reshape(2, 1, DIM)
    gate_pair = jnp.stack([gate_entity, gate_relation]).reshape(2, 1, DIM)

    bb = min(128, batch)
    o_out = _branches(o_stack, adj_stack, a_stack, w_pair, v_pair, gate_pair,
                      bb=bb)
    p_score, n_score = _scores(o_out, bb=min(1024, batch))
    return p_score, n_score


# speedup vs baseline: 4.3088x; 2.3951x over previous
import functools

import jax
import jax.numpy as jnp
from jax import lax
from jax.experimental import pallas as pl
from jax.experimental.pallas import tpu as pltpu
from jax.experimental.pallas import tpu_sc as plsc

DIM = 128
C = 32
CP1 = C + 1

_INTERPRET = False

_NC = 2
_NS = 16
_NW = _NC * _NS
_CH = 128


def _sc_stage(idx_hbm, tab, out_hbm, span, out_base, pair, wid,
              idx_v, rows_v, sems, sum_v):
    base = wid * span
    nch = span // _CH
    obase = out_base + (base // 2 if pair else base)

    def fire(c, b):
        off = pl.multiple_of(base + c * _CH, _CH)
        pltpu.sync_copy(idx_hbm.at[pl.ds(off, _CH)], idx_v[b])
        pltpu.async_copy(tab.at[idx_v[b]], rows_v[b], sems[b])

    def drain(b):
        pltpu.make_async_copy(tab.at[idx_v[b]], rows_v[b], sems[b]).wait()

    def write(c, b):
        if pair:
            half = _CH // 2

            def psum(k, carry):
                for d in range(DIM // 16):
                    sl = pl.ds(d * 16, 16)
                    sum_v[k, sl] = rows_v[b][2 * k, sl] + rows_v[b][2 * k + 1, sl]
                return carry

            lax.fori_loop(0, half, psum, 0)
            ooff = pl.multiple_of(obase + c * half, half)
            pltpu.sync_copy(sum_v, out_hbm.at[pl.ds(ooff, half)])
        else:
            ooff = pl.multiple_of(obase + c * _CH, _CH)
            pltpu.sync_copy(rows_v[b], out_hbm.at[pl.ds(ooff, _CH)])

    fire(0, 0)

    def body(i, carry):
        for b in range(2):
            c = 2 * i + b

            @pl.when(c + 1 < nch)
            def _():
                fire(c + 1, (b + 1) % 2)

            drain(b)
            write(c, b)
        return carry

    lax.fori_loop(0, nch // 2, body, 0)


def _sc_gather_body(ectx_idx, rctx_idx, eemb_idx, remb_idx,
                    etab, rtab, eemb, remb,
                    adj_out, o_out,
                    idx0, idx1, rows0, rows1, sum_v, sem0, sem1):
    wid = lax.axis_index("s") * _NC + lax.axis_index("c")
    idx_v = (idx0, idx1)
    rows_v = (rows0, rows1)
    sems = (sem0, sem1)
    n_e = ectx_idx.shape[0]
    n_r = rctx_idx.shape[0]
    n_oe = eemb_idx.shape[0]
    n_or = remb_idx.shape[0]
    _sc_stage(ectx_idx, etab, adj_out, n_e // _NW, 0, False, wid,
              idx_v, rows_v, sems, sum_v)
    _sc_stage(rctx_idx, rtab, adj_out, n_r // _NW, n_e, True, wid,
              idx_v, rows_v, sems, sum_v)
    _sc_stage(eemb_idx, eemb, o_out, n_oe // _NW, 0, False, wid,
              idx_v, rows_v, sems, sum_v)
    _sc_stage(remb_idx, remb, o_out, n_or // _NW, n_oe, False, wid,
              idx_v, rows_v, sems, sum_v)


def _sc_gather(ectx_idx, rctx_idx, eemb_idx, remb_idx,
               etab, rtab, eemb, remb):
    mesh = plsc.VectorSubcoreMesh(core_axis_name="c", subcore_axis_name="s")
    n_adj = ectx_idx.shape[0] + rctx_idx.shape[0] // 2
    n_o = eemb_idx.shape[0] + remb_idx.shape[0]
    f = pl.kernel(
        _sc_gather_body,
        out_type=[
            jax.ShapeDtypeStruct((n_adj, DIM), jnp.float32),
            jax.ShapeDtypeStruct((n_o, DIM), jnp.float32),
        ],
        mesh=mesh,
        scratch_types=[
            pltpu.VMEM((_CH,), jnp.int32),
            pltpu.VMEM((_CH,), jnp.int32),
            pltpu.VMEM((_CH, DIM), jnp.float32),
            pltpu.VMEM((_CH, DIM), jnp.float32),
            pltpu.VMEM((_CH // 2, DIM), jnp.float32),
            pltpu.SemaphoreType.DMA,
            pltpu.SemaphoreType.DMA,
        ],
    )
    return f(ectx_idx, rctx_idx, eemb_idx, remb_idx, etab, rtab, eemb, remb)


def _branch_math(o, adj, a3, w, v, gate, bb):
    o_w = jnp.dot(o, w, preferred_element_type=jnp.float32)
    adj_w = jnp.dot(adj.reshape(bb * C, DIM), w,
                    preferred_element_type=jnp.float32)
    vw3 = jnp.concatenate([o_w[:, None, :], adj_w.reshape(bb, C, DIM)],
                          axis=1)
    s3 = lax.dot_general(a3, vw3, (((2,), (1,)), ((0,), (0,))),
                         preferred_element_type=jnp.float32)
    h = jnp.maximum(s3, 0.0)
    tmp = jnp.maximum(h * o[:, None, :], 0.0)
    score = jnp.sum(tmp * v[None, None, :], axis=2)
    m = jnp.max(score, axis=1, keepdims=True)
    e = jnp.exp(score - m)
    alpha = e / jnp.sum(e, axis=1, keepdims=True)
    sg = jnp.sum(alpha[:, :, None] * h, axis=1)
    g = jax.nn.sigmoid(gate)
    return g[None, :] * o + (1.0 - g[None, :]) * sg


def _fused_body(o_ref, adj_ref, a_ref, w_ref, v_ref, gate_ref,
                dp_ref, dn_ref, acc_p, acc_n, *, bb):
    br = pl.program_id(1)
    out = _branch_math(o_ref[0], adj_ref[0], a_ref[0], w_ref[0],
                       v_ref[0, 0], gate_ref[0, 0], bb)

    @pl.when(br == 0)
    def _():
        acc_p[...] = out

    @pl.when(br == 1)
    def _():
        acc_p[...] = acc_p[...] - out

    @pl.when(br == 2)
    def _():
        acc_n[...] = out

    @pl.when(br == 3)
    def _():
        acc_n[...] = acc_n[...] - out

    @pl.when(br == 4)
    def _():
        dp_ref[0] = acc_p[...] + out

    @pl.when(br == 5)
    def _():
        dn_ref[0] = acc_n[...] + out


def _fused(o_all, adj_all, a_stack, w_pair, v_pair, gate_pair, *, bb):
    batch = o_all.shape[1]
    grid = (batch // bb, 6)
    return pl.pallas_call(
        functools.partial(_fused_body, bb=bb),
        grid=grid,
        in_specs=[
            pl.BlockSpec((1, bb, DIM), lambda g, b: (b, g, 0)),
            pl.BlockSpec((1, bb, C, DIM), lambda g, b: (b, g, 0, 0)),
            pl.BlockSpec((1, bb, CP1, CP1), lambda g, b: (b, g, 0, 0)),
            pl.BlockSpec((1, DIM, DIM), lambda g, b: (b // 4, 0, 0)),
            pl.BlockSpec((1, 1, DIM), lambda g, b: (b // 4, 0, 0)),
            pl.BlockSpec((1, 1, DIM), lambda g, b: (b // 4, 0, 0)),
        ],
        out_specs=[
            pl.BlockSpec((1, bb, DIM), lambda g, b: (0, g, 0)),
            pl.BlockSpec((1, bb, DIM), lambda g, b: (0, g, 0)),
        ],
        out_shape=[
            jax.ShapeDtypeStruct((1, batch, DIM), jnp.float32),
            jax.ShapeDtypeStruct((1, batch, DIM), jnp.float32),
        ],
        scratch_shapes=[
            pltpu.VMEM((bb, DIM), jnp.float32),
            pltpu.VMEM((bb, DIM), jnp.float32),
        ],
        interpret=_INTERPRET,
    )(o_all, adj_all, a_stack, w_pair, v_pair, gate_pair)


def _score_body(dp_ref, dn_ref, p_ref, n_ref):
    p_ref[...] = jnp.sum(jnp.abs(dp_ref[0]), axis=1)
    n_ref[...] = jnp.sum(jnp.abs(dn_ref[0]), axis=1)


def _scores(dp, dn, *, bs):
    batch = dp.shape[1]
    grid = (batch // bs,)
    return pl.pallas_call(
        _score_body,
        grid=grid,
        in_specs=[
            pl.BlockSpec((1, bs, DIM), lambda g: (0, g, 0)),
            pl.BlockSpec((1, bs, DIM), lambda g: (0, g, 0)),
        ],
        out_specs=[
            pl.BlockSpec((bs,), lambda g: (g,)),
            pl.BlockSpec((bs,), lambda g: (g,)),
        ],
        out_shape=[
            jax.ShapeDtypeStruct((batch,), jnp.float32),
            jax.ShapeDtypeStruct((batch,), jnp.float32),
        ],
        interpret=_INTERPRET,
    )(dp, dn)


def kernel(epoch, pos_h, pos_r, pos_t, neg_h, neg_r, neg_t,
           ph_A, pr_A, pt_A, nh_A, nr_A, nt_A,
           ph_ctx, pt_ctx, nh_ctx, nt_ctx, pr_ctx, nr_ctx,
           entity_emb, relation_emb, entity_context, relation_context,
           entity_gcn_weight, relation_gcn_weight,
           gate_entity, gate_relation, v_ent, v_rel):
    batch = pos_h.shape[0]

    i32 = jnp.int32
    ectx_idx = jnp.concatenate(
        [ph_ctx, pt_ctx, nh_ctx, nt_ctx], axis=0).astype(i32).reshape(-1)
    rctx_idx = jnp.concatenate(
        [pr_ctx, nr_ctx], axis=0).astype(i32).reshape(-1)
    eemb_idx = jnp.concatenate(
        [pos_h, pos_t, neg_h, neg_t], axis=0).astype(i32)
    remb_idx = jnp.concatenate([pos_r, neg_r], axis=0).astype(i32)

    adj_rows, o_rows = _sc_gather(
        ectx_idx, rctx_idx, eemb_idx, remb_idx,
        entity_context, relation_context, entity_emb, relation_emb)

    adj_all = adj_rows.reshape(6, batch, C, DIM)
    o_all = o_rows.reshape(6, batch, DIM)

    a_stack = jnp.stack([ph_A, pt_A, nh_A, nt_A, pr_A, nr_A])
    w_pair = jnp.stack([entity_gcn_weight, relation_gcn_weight])
    v_pair = jnp.stack([v_ent, v_rel]).reshape(2, 1, DIM)
    gate_pair = jnp.stack([gate_entity, gate_relation]).reshape(2, 1, DIM)

    bb = min(64, batch)
    dp, dn = _fused(o_all, adj_all, a_stack, w_pair, v_pair, gate_pair, bb=bb)
    p_score, n_score = _scores(dp, dn, bs=min(1024, batch))
    return p_score, n_score
